# baseline (speedup 1.0000x reference)
import functools
import jax
import jax.numpy as jnp
from jax import lax
from jax.experimental import pallas as pl
from jax.experimental.pallas import tpu as pltpu
from jax.experimental.pallas import tpu_sc as plsc

E = 64
TOP_K = 2
H = 1024
I = 1024
S = 2048
T = 64
NC = 2
NS = 16
NW = NC * NS
TPW = S // NW
PADTOT = 4672


def _router_body(x_ref, gw_ref, eb_ref, dst_ref, oc_ref, wc_ref, ut_ref):
    logits = jax.lax.dot_general(
        gw_ref[...], x_ref[...],
        dimension_numbers=(((1,), (1,)), ((), ())),
        preferred_element_type=jnp.float32)
    rw = jax.nn.sigmoid(logits)
    scores = rw + eb_ref[:, 0:1]
    esub = jax.lax.broadcasted_iota(jnp.int32, (E, S), 0)
    m1 = jnp.max(scores, axis=0, keepdims=True)
    a1 = jnp.min(jnp.where(scores == m1, esub, E), axis=0, keepdims=True)
    sel1 = esub == a1
    masked = jnp.where(sel1, -jnp.inf, scores)
    m2 = jnp.max(masked, axis=0, keepdims=True)
    a2 = jnp.min(jnp.where(masked == m2, esub, E), axis=0, keepdims=True)
    sel2 = esub == a2

    C = 512
    for j0 in range(0, S, C):
        ii = jax.lax.broadcasted_iota(jnp.int32, (S, C), 0)
        jj = jax.lax.broadcasted_iota(jnp.int32, (S, C), 1) + j0
        ut_ref[:, j0:j0 + C] = (ii <= jj).astype(jnp.float32)

    self_f = (sel1 | sel2).astype(jnp.float32)
    rank = jax.lax.dot_general(
        self_f, ut_ref[...], dimension_numbers=(((1,), (0,)), ((), ())),
        preferred_element_type=jnp.float32) - 1.0

    ones_row = jnp.ones((1, S), jnp.float32)
    cnt_row = jax.lax.dot_general(
        ones_row, self_f, dimension_numbers=(((1,), (1,)), ((), ())),
        preferred_element_type=jnp.float32)
    pcnt = (((cnt_row.astype(jnp.int32) + 7) // 8) * 8).astype(jnp.float32)
    ei = jax.lax.broadcasted_iota(jnp.int32, (E, E), 0)
    ej = jax.lax.broadcasted_iota(jnp.int32, (E, E), 1)
    slt = (ei < ej).astype(jnp.float32)
    offs_row = jax.lax.dot_general(
        pcnt, slt, dimension_numbers=(((1,), (0,)), ((), ())),
        preferred_element_type=jnp.float32)

    sel1f = sel1.astype(jnp.float32)
    sel2f = sel2.astype(jnp.float32)
    og1 = jax.lax.dot_general(
        offs_row, sel1f, dimension_numbers=(((1,), (0,)), ((), ())),
        preferred_element_type=jnp.float32,
        precision=jax.lax.Precision.HIGHEST)
    og2 = jax.lax.dot_general(
        offs_row, sel2f, dimension_numbers=(((1,), (0,)), ((), ())),
        preferred_element_type=jnp.float32,
        precision=jax.lax.Precision.HIGHEST)
    r1 = jnp.sum(jnp.where(sel1, rank, 0.0), axis=0, keepdims=True)
    r2 = jnp.sum(jnp.where(sel2, rank, 0.0), axis=0, keepdims=True)
    dest1 = og1 + r1
    dest2 = og2 + r2

    di8 = jax.lax.broadcasted_iota(jnp.int32, (8, S), 0)
    dst_ref[...] = jnp.where(
        di8 == 0, dest1, jnp.where(di8 == 1, dest2, 0.0)).astype(jnp.int32)
    de8 = jax.lax.broadcasted_iota(jnp.int32, (8, E), 0)
    oc_ref[...] = jnp.where(
        de8 == 0, offs_row, jnp.where(de8 == 1, cnt_row, 0.0)).astype(jnp.int32)

    ones_e = jnp.ones((E, 128), jnp.float32)
    w1col = jax.lax.dot_general(
        jnp.where(sel1, rw, 0.0), ones_e,
        dimension_numbers=(((0,), (0,)), ((), ())),
        preferred_element_type=jnp.float32)
    w2col = jax.lax.dot_general(
        jnp.where(sel2, rw, 0.0), ones_e,
        dimension_numbers=(((0,), (0,)), ((), ())),
        preferred_element_type=jnp.float32)
    ssum = w1col + w2col
    wc_ref[:, 0:128] = w1col / ssum
    wc_ref[:, 128:256] = w2col / ssum


def _router(x, gate_w, eb):
    return pl.pallas_call(
        _router_body,
        out_shape=(
            jax.ShapeDtypeStruct((8, S), jnp.int32),
            jax.ShapeDtypeStruct((8, E), jnp.int32),
            jax.ShapeDtypeStruct((S, 256), jnp.float32),
        ),
        in_specs=[
            pl.BlockSpec((S, H), lambda: (0, 0)),
            pl.BlockSpec((E, H), lambda: (0, 0)),
            pl.BlockSpec((E, 128), lambda: (0, 0)),
        ],
        out_specs=(
            pl.BlockSpec((8, S), lambda: (0, 0)),
            pl.BlockSpec((8, E), lambda: (0, 0)),
            pl.BlockSpec((S, 256), lambda: (0, 0)),
        ),
        scratch_shapes=[pltpu.VMEM((S, S), jnp.float32)],
    )(x, gate_w, eb)


def _make_dispatch():
    mesh = plsc.VectorSubcoreMesh(core_axis_name="c", subcore_axis_name="s")

    @functools.partial(
        pl.kernel, mesh=mesh,
        out_type=jax.ShapeDtypeStruct((PADTOT, H // 2), jnp.int32),
        scratch_types=[
            pltpu.VMEM((TPW, H // 2), jnp.int32),
            pltpu.VMEM((TPW,), jnp.int32),
            pltpu.VMEM((TPW,), jnp.int32),
            pltpu.SemaphoreType.DMA,
            pltpu.SemaphoreType.DMA,
        ])
    def disp(xb_hbm, d1_hbm, d2_hbm, xs_hbm, rows_v, i1_v, i2_v, s0, s1):
        wid = lax.axis_index("s") * NC + lax.axis_index("c")
        base = wid * TPW
        pltpu.sync_copy(xb_hbm.at[pl.ds(base, TPW)], rows_v)
        pltpu.sync_copy(d1_hbm.at[pl.ds(base, TPW)], i1_v)
        pltpu.sync_copy(d2_hbm.at[pl.ds(base, TPW)], i2_v)
        c0 = pltpu.async_copy(rows_v, xs_hbm.at[i1_v], s0)
        c1 = pltpu.async_copy(rows_v, xs_hbm.at[i2_v], s1)
        c0.wait()
        c1.wait()

    return disp


def _ffn_body(oc_ref, xs_ref, W1_ref, W2_ref, W3_ref, ys_ref):
    e = pl.program_id(0)
    off = oc_ref[0, e]
    cnt = oc_ref[1, e]
    trips = (cnt + T - 1) // T
    W1 = W1_ref[0]
    W2 = W2_ref[0]
    W3 = W3_ref[0]

    def tile_body(t, carry):
        base = pl.multiple_of(off + t * T, 8)
        xsb = xs_ref[pl.ds(base, T), :]
        a = jax.lax.dot_general(
            xsb, W1, dimension_numbers=(((1,), (1,)), ((), ())),
            preferred_element_type=jnp.float32)
        bb = jax.lax.dot_general(
            xsb, W3, dimension_numbers=(((1,), (1,)), ((), ())),
            preferred_element_type=jnp.float32)
        hh = (a * jax.nn.sigmoid(a) * bb).astype(jnp.bfloat16)
        y = jax.lax.dot_general(
            hh, W2, dimension_numbers=(((1,), (1,)), ((), ())),
            preferred_element_type=jnp.float32)
        ys_ref[pl.ds(base, T), :] = y
        return carry

    jax.lax.fori_loop(0, trips, tile_body, 0)


def _ffn(oc, xs, W1b, W2b, W3b):
    return pl.pallas_call(
        _ffn_body,
        grid=(E,),
        out_shape=jax.ShapeDtypeStruct((PADTOT, H), jnp.float32),
        in_specs=[
            pl.BlockSpec(memory_space=pltpu.SMEM),
            pl.BlockSpec((PADTOT, H), lambda e: (0, 0)),
            pl.BlockSpec((1, I, H), lambda e: (e, 0, 0)),
            pl.BlockSpec((1, H, I), lambda e: (e, 0, 0)),
            pl.BlockSpec((1, I, H), lambda e: (e, 0, 0)),
        ],
        out_specs=pl.BlockSpec((PADTOT, H), lambda e: (0, 0)),
        compiler_params=pltpu.CompilerParams(
            dimension_semantics=("arbitrary",),
            vmem_limit_bytes=66060288),
    )(oc, xs, W1b, W2b, W3b)


def _make_gather():
    mesh = plsc.VectorSubcoreMesh(core_axis_name="c", subcore_axis_name="s")

    @functools.partial(
        pl.kernel, mesh=mesh,
        out_type=[
            jax.ShapeDtypeStruct((S, H), jnp.float32),
            jax.ShapeDtypeStruct((S, H), jnp.float32),
        ],
        scratch_types=[
            pltpu.VMEM((TPW,), jnp.int32),
            pltpu.VMEM((TPW,), jnp.int32),
            pltpu.VMEM((TPW, H), jnp.float32),
            pltpu.SemaphoreType.DMA,
        ])
    def gath(ys_hbm, d1_hbm, d2_hbm, g1_hbm, g2_hbm, i1_v, i2_v, buf_v, s0):
        wid = lax.axis_index("s") * NC + lax.axis_index("c")
        base = wid * TPW
        pltpu.sync_copy(d1_hbm.at[pl.ds(base, TPW)], i1_v)
        pltpu.sync_copy(d2_hbm.at[pl.ds(base, TPW)], i2_v)
        pltpu.async_copy(ys_hbm.at[i1_v], buf_v, s0).wait()
        pltpu.sync_copy(buf_v, g1_hbm.at[pl.ds(base, TPW)])
        pltpu.async_copy(ys_hbm.at[i2_v], buf_v, s0).wait()
        pltpu.sync_copy(buf_v, g2_hbm.at[pl.ds(base, TPW)])

    return gath


def _combine_body(g1_ref, g2_ref, wc_ref, out_ref):
    w1 = wc_ref[:, 0:1]
    w2 = wc_ref[:, 128:129]
    out_ref[...] = g1_ref[...] * w1 + g2_ref[...] * w2


def _combine(g1, g2, wc):
    return pl.pallas_call(
        _combine_body,
        out_shape=jax.ShapeDtypeStruct((S, H), jnp.float32),
        in_specs=[
            pl.BlockSpec((S, H), lambda: (0, 0)),
            pl.BlockSpec((S, H), lambda: (0, 0)),
            pl.BlockSpec((S, 256), lambda: (0, 0)),
        ],
        out_specs=pl.BlockSpec((S, H), lambda: (0, 0)),
    )(g1, g2, wc)


def kernel(hidden_states, gate_w, e_bias, W1, W2, W3):
    b, s, h = hidden_states.shape
    x = hidden_states.reshape(s, h)
    eb = jnp.broadcast_to(e_bias[:, None], (E, 128))

    dests, oc, wc = _router(x, gate_w, eb)
    d1 = dests[0]
    d2 = dests[1]

    xb = x.astype(jnp.bfloat16)
    xbi = jax.lax.bitcast_convert_type(
        xb.reshape(S, H // 2, 2), jnp.int32)
    xsi = _make_dispatch()(xbi, d1, d2)
    xs = jax.lax.bitcast_convert_type(
        xsi, jnp.bfloat16).reshape(PADTOT, H)

    W1b = W1.astype(jnp.bfloat16)
    W2b = W2.astype(jnp.bfloat16)
    W3b = W3.astype(jnp.bfloat16)
    ys = _ffn(oc, xs, W1b, W2b, W3b)

    g1, g2 = _make_gather()(ys, d1, d2)
    out = _combine(g1, g2, wc)
    return out.reshape(b, s, h)


# speedup vs baseline: 1.1571x; 1.1571x over previous
import functools
import jax
import jax.numpy as jnp
from jax import lax
from jax.experimental import pallas as pl
from jax.experimental.pallas import tpu as pltpu
from jax.experimental.pallas import tpu_sc as plsc

E = 64
TOP_K = 2
H = 1024
I = 1024
S = 2048
T = 64
NC = 2
NS = 16
NW = NC * NS
TPW = S // NW
PADTOT = 4608


def _router_body(x_ref, gw_ref, eb_ref, dst_ref, oc_ref, wc_ref, ut_ref):
    logits = jax.lax.dot_general(
        gw_ref[...], x_ref[...],
        dimension_numbers=(((1,), (1,)), ((), ())),
        preferred_element_type=jnp.float32)
    rw = jax.nn.sigmoid(logits)
    scores = rw + eb_ref[:, 0:1]
    esub = jax.lax.broadcasted_iota(jnp.int32, (E, S), 0)
    m1 = jnp.max(scores, axis=0, keepdims=True)
    a1 = jnp.min(jnp.where(scores == m1, esub, E), axis=0, keepdims=True)
    sel1 = esub == a1
    masked = jnp.where(sel1, -jnp.inf, scores)
    m2 = jnp.max(masked, axis=0, keepdims=True)
    a2 = jnp.min(jnp.where(masked == m2, esub, E), axis=0, keepdims=True)
    sel2 = esub == a2

    C = 512
    for j0 in range(0, S, C):
        ii = jax.lax.broadcasted_iota(jnp.int32, (S, C), 0)
        jj = jax.lax.broadcasted_iota(jnp.int32, (S, C), 1) + j0
        ut_ref[:, j0:j0 + C] = (ii <= jj).astype(jnp.float32)

    self_f = (sel1 | sel2).astype(jnp.float32)
    rank = jax.lax.dot_general(
        self_f, ut_ref[...], dimension_numbers=(((1,), (0,)), ((), ())),
        preferred_element_type=jnp.float32) - 1.0

    ones_row = jnp.ones((1, S), jnp.float32)
    cnt_row = jax.lax.dot_general(
        ones_row, self_f, dimension_numbers=(((1,), (1,)), ((), ())),
        preferred_element_type=jnp.float32)
    pcnt = (((cnt_row.astype(jnp.int32) + 7) // 8) * 8).astype(jnp.float32)
    ei = jax.lax.broadcasted_iota(jnp.int32, (E, E), 0)
    ej = jax.lax.broadcasted_iota(jnp.int32, (E, E), 1)
    slt = (ei < ej).astype(jnp.float32)
    offs_row = jax.lax.dot_general(
        pcnt, slt, dimension_numbers=(((1,), (0,)), ((), ())),
        preferred_element_type=jnp.float32)

    sel1f = sel1.astype(jnp.float32)
    sel2f = sel2.astype(jnp.float32)
    og1 = jax.lax.dot_general(
        offs_row, sel1f, dimension_numbers=(((1,), (0,)), ((), ())),
        preferred_element_type=jnp.float32,
        precision=jax.lax.Precision.HIGHEST)
    og2 = jax.lax.dot_general(
        offs_row, sel2f, dimension_numbers=(((1,), (0,)), ((), ())),
        preferred_element_type=jnp.float32,
        precision=jax.lax.Precision.HIGHEST)
    r1 = jnp.sum(jnp.where(sel1, rank, 0.0), axis=0, keepdims=True)
    r2 = jnp.sum(jnp.where(sel2, rank, 0.0), axis=0, keepdims=True)
    dest1 = og1 + r1
    dest2 = og2 + r2

    di8 = jax.lax.broadcasted_iota(jnp.int32, (8, S), 0)
    dst_ref[...] = jnp.where(
        di8 == 0, dest1, jnp.where(di8 == 1, dest2, 0.0)).astype(jnp.int32)
    de8 = jax.lax.broadcasted_iota(jnp.int32, (8, E), 0)
    oc_ref[...] = jnp.where(
        de8 == 0, offs_row, jnp.where(de8 == 1, cnt_row, 0.0)).astype(jnp.int32)

    ones_e = jnp.ones((E, 128), jnp.float32)
    w1col = jax.lax.dot_general(
        jnp.where(sel1, rw, 0.0), ones_e,
        dimension_numbers=(((0,), (0,)), ((), ())),
        preferred_element_type=jnp.float32)
    w2col = jax.lax.dot_general(
        jnp.where(sel2, rw, 0.0), ones_e,
        dimension_numbers=(((0,), (0,)), ((), ())),
        preferred_element_type=jnp.float32)
    ssum = w1col + w2col
    wc_ref[:, 0:128] = w1col / ssum
    wc_ref[:, 128:256] = w2col / ssum


def _router(x, gate_w, eb):
    return pl.pallas_call(
        _router_body,
        out_shape=(
            jax.ShapeDtypeStruct((8, S), jnp.int32),
            jax.ShapeDtypeStruct((8, E), jnp.int32),
            jax.ShapeDtypeStruct((S, 256), jnp.float32),
        ),
        in_specs=[
            pl.BlockSpec((S, H), lambda: (0, 0)),
            pl.BlockSpec((E, H), lambda: (0, 0)),
            pl.BlockSpec((E, 128), lambda: (0, 0)),
        ],
        out_specs=(
            pl.BlockSpec((8, S), lambda: (0, 0)),
            pl.BlockSpec((8, E), lambda: (0, 0)),
            pl.BlockSpec((S, 256), lambda: (0, 0)),
        ),
        scratch_shapes=[pltpu.VMEM((S, S), jnp.float32)],
    )(x, gate_w, eb)


def _make_dispatch():
    mesh = plsc.VectorSubcoreMesh(core_axis_name="c", subcore_axis_name="s")

    @functools.partial(
        pl.kernel, mesh=mesh,
        out_type=jax.ShapeDtypeStruct((PADTOT, H // 2), jnp.int32),
        scratch_types=[
            pltpu.VMEM((TPW, H // 2), jnp.int32),
            pltpu.VMEM((TPW,), jnp.int32),
            pltpu.VMEM((TPW,), jnp.int32),
            pltpu.SemaphoreType.DMA,
            pltpu.SemaphoreType.DMA,
        ])
    def disp(xb_hbm, d1_hbm, d2_hbm, xs_hbm, rows_v, i1_v, i2_v, s0, s1):
        wid = lax.axis_index("s") * NC + lax.axis_index("c")
        base = wid * TPW
        pltpu.sync_copy(xb_hbm.at[pl.ds(base, TPW)], rows_v)
        pltpu.sync_copy(d1_hbm.at[pl.ds(base, TPW)], i1_v)
        pltpu.sync_copy(d2_hbm.at[pl.ds(base, TPW)], i2_v)
        c0 = pltpu.async_copy(rows_v, xs_hbm.at[i1_v], s0)
        c1 = pltpu.async_copy(rows_v, xs_hbm.at[i2_v], s1)
        c0.wait()
        c1.wait()

    return disp


def _ffn_body(oc_ref, xs_ref, W1_ref, W2_ref, W3_ref, ys_ref):
    e = pl.program_id(0)
    off = oc_ref[0, e]
    cnt = oc_ref[1, e]
    trips = (cnt + T - 1) // T
    W1 = W1_ref[0]
    W2 = W2_ref[0]
    W3 = W3_ref[0]

    def tile_body(t, carry):
        base = pl.multiple_of(off + t * T, 8)
        xsf = xs_ref[pl.ds(base, T), :].astype(jnp.float32)
        a = jax.lax.dot_general(
            xsf, W1, dimension_numbers=(((1,), (1,)), ((), ())),
            preferred_element_type=jnp.float32)
        bb = jax.lax.dot_general(
            xsf, W3, dimension_numbers=(((1,), (1,)), ((), ())),
            preferred_element_type=jnp.float32)
        hh = a * jax.nn.sigmoid(a) * bb
        y = jax.lax.dot_general(
            hh, W2, dimension_numbers=(((1,), (1,)), ((), ())),
            preferred_element_type=jnp.float32)
        ys_ref[pl.ds(base, T), :] = y.astype(jnp.bfloat16)
        return carry

    jax.lax.fori_loop(0, trips, tile_body, 0)


def _ffn(oc, xs, W1b, W2b, W3b):
    return pl.pallas_call(
        _ffn_body,
        grid=(E,),
        out_shape=jax.ShapeDtypeStruct((PADTOT, H), jnp.bfloat16),
        in_specs=[
            pl.BlockSpec(memory_space=pltpu.SMEM),
            pl.BlockSpec((PADTOT, H), lambda e: (0, 0)),
            pl.BlockSpec((1, I, H), lambda e: (e, 0, 0)),
            pl.BlockSpec((1, H, I), lambda e: (e, 0, 0)),
            pl.BlockSpec((1, I, H), lambda e: (e, 0, 0)),
        ],
        out_specs=pl.BlockSpec((PADTOT, H), lambda e: (0, 0)),
        compiler_params=pltpu.CompilerParams(
            dimension_semantics=("arbitrary",),
            vmem_limit_bytes=66060288),
    )(oc, xs, W1b, W2b, W3b)


def _make_gather():
    mesh = plsc.VectorSubcoreMesh(core_axis_name="c", subcore_axis_name="s")

    @functools.partial(
        pl.kernel, mesh=mesh,
        out_type=[
            jax.ShapeDtypeStruct((S, H // 2), jnp.int32),
            jax.ShapeDtypeStruct((S, H // 2), jnp.int32),
        ],
        scratch_types=[
            pltpu.VMEM((TPW,), jnp.int32),
            pltpu.VMEM((TPW,), jnp.int32),
            pltpu.VMEM((TPW, H // 2), jnp.int32),
            pltpu.SemaphoreType.DMA,
        ])
    def gath(ys_hbm, d1_hbm, d2_hbm, g1_hbm, g2_hbm, i1_v, i2_v, buf_v, s0):
        wid = lax.axis_index("s") * NC + lax.axis_index("c")
        base = wid * TPW
        pltpu.sync_copy(d1_hbm.at[pl.ds(base, TPW)], i1_v)
        pltpu.sync_copy(d2_hbm.at[pl.ds(base, TPW)], i2_v)
        pltpu.async_copy(ys_hbm.at[i1_v], buf_v, s0).wait()
        pltpu.sync_copy(buf_v, g1_hbm.at[pl.ds(base, TPW)])
        pltpu.async_copy(ys_hbm.at[i2_v], buf_v, s0).wait()
        pltpu.sync_copy(buf_v, g2_hbm.at[pl.ds(base, TPW)])

    return gath


def _combine_body(g1_ref, g2_ref, wc_ref, out_ref):
    w1 = wc_ref[:, 0:1]
    w2 = wc_ref[:, 128:129]
    g1 = g1_ref[...].astype(jnp.float32)
    g2 = g2_ref[...].astype(jnp.float32)
    out_ref[...] = g1 * w1 + g2 * w2


def _combine(g1, g2, wc):
    return pl.pallas_call(
        _combine_body,
        out_shape=jax.ShapeDtypeStruct((S, H), jnp.float32),
        in_specs=[
            pl.BlockSpec((S, H), lambda: (0, 0)),
            pl.BlockSpec((S, H), lambda: (0, 0)),
            pl.BlockSpec((S, 256), lambda: (0, 0)),
        ],
        out_specs=pl.BlockSpec((S, H), lambda: (0, 0)),
    )(g1, g2, wc)


def kernel(hidden_states, gate_w, e_bias, W1, W2, W3):
    b, s, h = hidden_states.shape
    x = hidden_states.reshape(s, h)
    eb = jnp.broadcast_to(e_bias[:, None], (E, 128))

    dests, oc, wc = _router(x, gate_w, eb)
    d1 = dests[0]
    d2 = dests[1]

    xb = x.astype(jnp.bfloat16)
    xbi = jax.lax.bitcast_convert_type(
        xb.reshape(S, H // 2, 2), jnp.int32)
    xsi = _make_dispatch()(xbi, d1, d2)
    xs = jax.lax.bitcast_convert_type(
        xsi, jnp.bfloat16).reshape(PADTOT, H)

    ys = _ffn(oc, xs, W1, W2, W3)
    ysi = jax.lax.bitcast_convert_type(
        ys.reshape(PADTOT, H // 2, 2), jnp.int32)

    g1i, g2i = _make_gather()(ysi, d1, d2)
    g1 = jax.lax.bitcast_convert_type(g1i, jnp.bfloat16).reshape(S, H)
    g2 = jax.lax.bitcast_convert_type(g2i, jnp.bfloat16).reshape(S, H)
    out = _combine(g1, g2, wc)
    return out.reshape(b, s, h)


# speedup vs baseline: 1.1963x; 1.0339x over previous
import functools
import jax
import jax.numpy as jnp
from jax import lax
from jax.experimental import pallas as pl
from jax.experimental.pallas import tpu as pltpu
from jax.experimental.pallas import tpu_sc as plsc

E = 64
TOP_K = 2
H = 1024
I = 1024
S = 2048
T = 64
NC = 2
NS = 16
NW = NC * NS
TPW = S // NW
PADTOT = 4608


def _router_body(x_ref, gw_ref, eb_ref, dst_ref, oc_ref, wc_ref, ut_ref):
    logits = jax.lax.dot_general(
        gw_ref[...], x_ref[...],
        dimension_numbers=(((1,), (1,)), ((), ())),
        preferred_element_type=jnp.float32)
    rw = jax.nn.sigmoid(logits)
    scores = rw + eb_ref[:, 0:1]
    esub = jax.lax.broadcasted_iota(jnp.int32, (E, S), 0)
    m1 = jnp.max(scores, axis=0, keepdims=True)
    a1 = jnp.min(jnp.where(scores == m1, esub, E), axis=0, keepdims=True)
    sel1 = esub == a1
    masked = jnp.where(sel1, -jnp.inf, scores)
    m2 = jnp.max(masked, axis=0, keepdims=True)
    a2 = jnp.min(jnp.where(masked == m2, esub, E), axis=0, keepdims=True)
    sel2 = esub == a2

    C = 512
    for j0 in range(0, S, C):
        ii = jax.lax.broadcasted_iota(jnp.int32, (S, C), 0)
        jj = jax.lax.broadcasted_iota(jnp.int32, (S, C), 1) + j0
        ut_ref[:, j0:j0 + C] = (ii <= jj).astype(jnp.float32)

    self_f = (sel1 | sel2).astype(jnp.float32)
    rank = jax.lax.dot_general(
        self_f, ut_ref[...], dimension_numbers=(((1,), (0,)), ((), ())),
        preferred_element_type=jnp.float32) - 1.0

    ones_row = jnp.ones((1, S), jnp.float32)
    cnt_row = jax.lax.dot_general(
        ones_row, self_f, dimension_numbers=(((1,), (1,)), ((), ())),
        preferred_element_type=jnp.float32)
    pcnt = (((cnt_row.astype(jnp.int32) + 7) // 8) * 8).astype(jnp.float32)
    ei = jax.lax.broadcasted_iota(jnp.int32, (E, E), 0)
    ej = jax.lax.broadcasted_iota(jnp.int32, (E, E), 1)
    slt = (ei < ej).astype(jnp.float32)
    offs_row = jax.lax.dot_general(
        pcnt, slt, dimension_numbers=(((1,), (0,)), ((), ())),
        preferred_element_type=jnp.float32)

    sel1f = sel1.astype(jnp.float32)
    sel2f = sel2.astype(jnp.float32)
    og1 = jax.lax.dot_general(
        offs_row, sel1f, dimension_numbers=(((1,), (0,)), ((), ())),
        preferred_element_type=jnp.float32,
        precision=jax.lax.Precision.HIGHEST)
    og2 = jax.lax.dot_general(
        offs_row, sel2f, dimension_numbers=(((1,), (0,)), ((), ())),
        preferred_element_type=jnp.float32,
        precision=jax.lax.Precision.HIGHEST)
    r1 = jnp.sum(jnp.where(sel1, rank, 0.0), axis=0, keepdims=True)
    r2 = jnp.sum(jnp.where(sel2, rank, 0.0), axis=0, keepdims=True)
    dest1 = og1 + r1
    dest2 = og2 + r2

    di8 = jax.lax.broadcasted_iota(jnp.int32, (8, S), 0)
    dst_ref[...] = jnp.where(
        di8 == 0, dest1, jnp.where(di8 == 1, dest2, 0.0)).astype(jnp.int32)
    de8 = jax.lax.broadcasted_iota(jnp.int32, (8, E), 0)
    oc_ref[...] = jnp.where(
        de8 == 0, offs_row, jnp.where(de8 == 1, cnt_row, 0.0)).astype(jnp.int32)

    ones_e = jnp.ones((E, 128), jnp.float32)
    w1col = jax.lax.dot_general(
        jnp.where(sel1, rw, 0.0), ones_e,
        dimension_numbers=(((0,), (0,)), ((), ())),
        preferred_element_type=jnp.float32)
    w2col = jax.lax.dot_general(
        jnp.where(sel2, rw, 0.0), ones_e,
        dimension_numbers=(((0,), (0,)), ((), ())),
        preferred_element_type=jnp.float32)
    ssum = w1col + w2col
    wc_ref[:, 0:128] = w1col / ssum
    wc_ref[:, 128:256] = w2col / ssum


def _router(x, gate_w, eb):
    return pl.pallas_call(
        _router_body,
        out_shape=(
            jax.ShapeDtypeStruct((8, S), jnp.int32),
            jax.ShapeDtypeStruct((8, E), jnp.int32),
            jax.ShapeDtypeStruct((S, 256), jnp.float32),
        ),
        in_specs=[
            pl.BlockSpec((S, H), lambda: (0, 0)),
            pl.BlockSpec((E, H), lambda: (0, 0)),
            pl.BlockSpec((E, 128), lambda: (0, 0)),
        ],
        out_specs=(
            pl.BlockSpec((8, S), lambda: (0, 0)),
            pl.BlockSpec((8, E), lambda: (0, 0)),
            pl.BlockSpec((S, 256), lambda: (0, 0)),
        ),
        scratch_shapes=[pltpu.VMEM((S, S), jnp.float32)],
    )(x, gate_w, eb)


def _make_dispatch():
    mesh = plsc.VectorSubcoreMesh(core_axis_name="c", subcore_axis_name="s")

    @functools.partial(
        pl.kernel, mesh=mesh,
        out_type=jax.ShapeDtypeStruct((PADTOT, H // 2), jnp.int32),
        scratch_types=[
            pltpu.VMEM((TPW, H // 2), jnp.int32),
            pltpu.VMEM((TPW,), jnp.int32),
            pltpu.VMEM((TPW,), jnp.int32),
            pltpu.SemaphoreType.DMA,
            pltpu.SemaphoreType.DMA,
        ])
    def disp(xb_hbm, d1_hbm, d2_hbm, xs_hbm, rows_v, i1_v, i2_v, s0, s1):
        wid = lax.axis_index("s") * NC + lax.axis_index("c")
        base = wid * TPW
        pltpu.sync_copy(xb_hbm.at[pl.ds(base, TPW)], rows_v)
        pltpu.sync_copy(d1_hbm.at[pl.ds(base, TPW)], i1_v)
        pltpu.sync_copy(d2_hbm.at[pl.ds(base, TPW)], i2_v)
        c0 = pltpu.async_copy(rows_v, xs_hbm.at[i1_v], s0)
        c1 = pltpu.async_copy(rows_v, xs_hbm.at[i2_v], s1)
        c0.wait()
        c1.wait()

    return disp


def _ffn_body(oc_ref, xs_ref, W1a_ref, W1b_ref, W2a_ref, W2b_ref,
              W3a_ref, W3b_ref, ys_ref):
    e = pl.program_id(0)
    off = oc_ref[0, e]
    cnt = oc_ref[1, e]
    trips = (cnt + T - 1) // T
    HI = I // 2

    def tile_body(t, carry):
        base = pl.multiple_of(off + t * T, 8)
        xsf = xs_ref[pl.ds(base, T), :].astype(jnp.float32)
        dn = (((1,), (1,)), ((), ()))
        a = jnp.concatenate([
            jax.lax.dot_general(xsf, W1a_ref[0], dimension_numbers=dn,
                                preferred_element_type=jnp.float32),
            jax.lax.dot_general(xsf, W1b_ref[0], dimension_numbers=dn,
                                preferred_element_type=jnp.float32),
        ], axis=1)
        bb = jnp.concatenate([
            jax.lax.dot_general(xsf, W3a_ref[0], dimension_numbers=dn,
                                preferred_element_type=jnp.float32),
            jax.lax.dot_general(xsf, W3b_ref[0], dimension_numbers=dn,
                                preferred_element_type=jnp.float32),
        ], axis=1)
        hh = a * jax.nn.sigmoid(a) * bb
        y = (jax.lax.dot_general(
                hh[:, :HI], W2a_ref[0], dimension_numbers=dn,
                preferred_element_type=jnp.float32)
             + jax.lax.dot_general(
                hh[:, HI:], W2b_ref[0], dimension_numbers=dn,
                preferred_element_type=jnp.float32))
        ys_ref[pl.ds(base, T), :] = y.astype(jnp.bfloat16)
        return carry

    jax.lax.fori_loop(0, trips, tile_body, 0)


def _ffn(oc, xs, W1, W2, W3):
    HI = I // 2
    return pl.pallas_call(
        _ffn_body,
        grid=(E,),
        out_shape=jax.ShapeDtypeStruct((PADTOT, H), jnp.bfloat16),
        in_specs=[
            pl.BlockSpec(memory_space=pltpu.SMEM),
            pl.BlockSpec((PADTOT, H), lambda e: (0, 0)),
            pl.BlockSpec((1, HI, H), lambda e: (e, 0, 0)),
            pl.BlockSpec((1, HI, H), lambda e: (e, 1, 0)),
            pl.BlockSpec((1, H, HI), lambda e: (e, 0, 0)),
            pl.BlockSpec((1, H, HI), lambda e: (e, 0, 1)),
            pl.BlockSpec((1, HI, H), lambda e: (e, 0, 0)),
            pl.BlockSpec((1, HI, H), lambda e: (e, 1, 0)),
        ],
        out_specs=pl.BlockSpec((PADTOT, H), lambda e: (0, 0)),
        compiler_params=pltpu.CompilerParams(
            dimension_semantics=("arbitrary",),
            vmem_limit_bytes=66060288),
    )(oc, xs, W1, W1, W2, W2, W3, W3)


def _make_gather():
    mesh = plsc.VectorSubcoreMesh(core_axis_name="c", subcore_axis_name="s")

    @functools.partial(
        pl.kernel, mesh=mesh,
        out_type=[
            jax.ShapeDtypeStruct((S, H // 2), jnp.int32),
            jax.ShapeDtypeStruct((S, H // 2), jnp.int32),
        ],
        scratch_types=[
            pltpu.VMEM((TPW,), jnp.int32),
            pltpu.VMEM((TPW,), jnp.int32),
            pltpu.VMEM((TPW, H // 2), jnp.int32),
            pltpu.SemaphoreType.DMA,
        ])
    def gath(ys_hbm, d1_hbm, d2_hbm, g1_hbm, g2_hbm, i1_v, i2_v, buf_v, s0):
        wid = lax.axis_index("s") * NC + lax.axis_index("c")
        base = wid * TPW
        pltpu.sync_copy(d1_hbm.at[pl.ds(base, TPW)], i1_v)
        pltpu.sync_copy(d2_hbm.at[pl.ds(base, TPW)], i2_v)
        pltpu.async_copy(ys_hbm.at[i1_v], buf_v, s0).wait()
        pltpu.sync_copy(buf_v, g1_hbm.at[pl.ds(base, TPW)])
        pltpu.async_copy(ys_hbm.at[i2_v], buf_v, s0).wait()
        pltpu.sync_copy(buf_v, g2_hbm.at[pl.ds(base, TPW)])

    return gath


def _combine_body(g1_ref, g2_ref, wc_ref, out_ref):
    w1 = wc_ref[:, 0:1]
    w2 = wc_ref[:, 128:129]
    g1 = g1_ref[...].astype(jnp.float32)
    g2 = g2_ref[...].astype(jnp.float32)
    out_ref[...] = g1 * w1 + g2 * w2


def _combine(g1, g2, wc):
    return pl.pallas_call(
        _combine_body,
        out_shape=jax.ShapeDtypeStruct((S, H), jnp.float32),
        in_specs=[
            pl.BlockSpec((S, H), lambda: (0, 0)),
            pl.BlockSpec((S, H), lambda: (0, 0)),
            pl.BlockSpec((S, 256), lambda: (0, 0)),
        ],
        out_specs=pl.BlockSpec((S, H), lambda: (0, 0)),
    )(g1, g2, wc)


def kernel(hidden_states, gate_w, e_bias, W1, W2, W3):
    b, s, h = hidden_states.shape
    x = hidden_states.reshape(s, h)
    eb = jnp.broadcast_to(e_bias[:, None], (E, 128))

    dests, oc, wc = _router(x, gate_w, eb)
    d1 = dests[0]
    d2 = dests[1]

    xb = x.astype(jnp.bfloat16)
    xbi = jax.lax.bitcast_convert_type(
        xb.reshape(S, H // 2, 2), jnp.int32)
    xsi = _make_dispatch()(xbi, d1, d2)
    xs = jax.lax.bitcast_convert_type(
        xsi, jnp.bfloat16).reshape(PADTOT, H)

    ys = _ffn(oc, xs, W1, W2, W3)
    ysi = jax.lax.bitcast_convert_type(
        ys.reshape(PADTOT, H // 2, 2), jnp.int32)

    g1i, g2i = _make_gather()(ysi, d1, d2)
    g1 = jax.lax.bitcast_convert_type(g1i, jnp.bfloat16).reshape(S, H)
    g2 = jax.lax.bitcast_convert_type(g2i, jnp.bfloat16).reshape(S, H)
    out = _combine(g1, g2, wc)
    return out.reshape(b, s, h)


# speedup vs baseline: 1.2360x; 1.0332x over previous
import functools
import jax
import jax.numpy as jnp
from jax import lax
from jax.experimental import pallas as pl
from jax.experimental.pallas import tpu as pltpu
from jax.experimental.pallas import tpu_sc as plsc

E = 64
TOP_K = 2
H = 1024
I = 1024
S = 2048
T = 128
NC = 2
NS = 16
NW = NC * NS
TPW = S // NW
PADTOT = 4672


def _router_body(x_ref, gw_ref, eb_ref, dst_ref, oc_ref, wc_ref, ut_ref):
    logits = jax.lax.dot_general(
        gw_ref[...], x_ref[...],
        dimension_numbers=(((1,), (1,)), ((), ())),
        preferred_element_type=jnp.float32)
    rw = jax.nn.sigmoid(logits)
    scores = rw + eb_ref[:, 0:1]
    esub = jax.lax.broadcasted_iota(jnp.int32, (E, S), 0)
    m1 = jnp.max(scores, axis=0, keepdims=True)
    a1 = jnp.min(jnp.where(scores == m1, esub, E), axis=0, keepdims=True)
    sel1 = esub == a1
    masked = jnp.where(sel1, -jnp.inf, scores)
    m2 = jnp.max(masked, axis=0, keepdims=True)
    a2 = jnp.min(jnp.where(masked == m2, esub, E), axis=0, keepdims=True)
    sel2 = esub == a2

    C = 512
    for j0 in range(0, S, C):
        ii = jax.lax.broadcasted_iota(jnp.int32, (S, C), 0)
        jj = jax.lax.broadcasted_iota(jnp.int32, (S, C), 1) + j0
        ut_ref[:, j0:j0 + C] = (ii <= jj).astype(jnp.float32)

    self_f = (sel1 | sel2).astype(jnp.float32)
    rank = jax.lax.dot_general(
        self_f, ut_ref[...], dimension_numbers=(((1,), (0,)), ((), ())),
        preferred_element_type=jnp.float32) - 1.0

    ones_row = jnp.ones((1, S), jnp.float32)
    cnt_row = jax.lax.dot_general(
        ones_row, self_f, dimension_numbers=(((1,), (1,)), ((), ())),
        preferred_element_type=jnp.float32)
    pcnt = (((cnt_row.astype(jnp.int32) + 7) // 8) * 8).astype(jnp.float32)
    ei = jax.lax.broadcasted_iota(jnp.int32, (E, E), 0)
    ej = jax.lax.broadcasted_iota(jnp.int32, (E, E), 1)
    slt = (ei < ej).astype(jnp.float32)
    offs_row = jax.lax.dot_general(
        pcnt, slt, dimension_numbers=(((1,), (0,)), ((), ())),
        preferred_element_type=jnp.float32)

    sel1f = sel1.astype(jnp.float32)
    sel2f = sel2.astype(jnp.float32)
    og1 = jax.lax.dot_general(
        offs_row, sel1f, dimension_numbers=(((1,), (0,)), ((), ())),
        preferred_element_type=jnp.float32,
        precision=jax.lax.Precision.HIGHEST)
    og2 = jax.lax.dot_general(
        offs_row, sel2f, dimension_numbers=(((1,), (0,)), ((), ())),
        preferred_element_type=jnp.float32,
        precision=jax.lax.Precision.HIGHEST)
    r1 = jnp.sum(jnp.where(sel1, rank, 0.0), axis=0, keepdims=True)
    r2 = jnp.sum(jnp.where(sel2, rank, 0.0), axis=0, keepdims=True)
    dest1 = og1 + r1
    dest2 = og2 + r2

    di8 = jax.lax.broadcasted_iota(jnp.int32, (8, S), 0)
    dst_ref[...] = jnp.where(
        di8 == 0, dest1, jnp.where(di8 == 1, dest2, 0.0)).astype(jnp.int32)
    de8 = jax.lax.broadcasted_iota(jnp.int32, (8, E), 0)
    oc_ref[...] = jnp.where(
        de8 == 0, offs_row, jnp.where(de8 == 1, cnt_row, 0.0)).astype(jnp.int32)

    ones_e = jnp.ones((E, 128), jnp.float32)
    w1col = jax.lax.dot_general(
        jnp.where(sel1, rw, 0.0), ones_e,
        dimension_numbers=(((0,), (0,)), ((), ())),
        preferred_element_type=jnp.float32)
    w2col = jax.lax.dot_general(
        jnp.where(sel2, rw, 0.0), ones_e,
        dimension_numbers=(((0,), (0,)), ((), ())),
        preferred_element_type=jnp.float32)
    ssum = w1col + w2col
    wc_ref[:, 0:128] = w1col / ssum
    wc_ref[:, 128:256] = w2col / ssum


def _router(x, gate_w, eb):
    return pl.pallas_call(
        _router_body,
        out_shape=(
            jax.ShapeDtypeStruct((8, S), jnp.int32),
            jax.ShapeDtypeStruct((8, E), jnp.int32),
            jax.ShapeDtypeStruct((S, 256), jnp.float32),
        ),
        in_specs=[
            pl.BlockSpec((S, H), lambda: (0, 0)),
            pl.BlockSpec((E, H), lambda: (0, 0)),
            pl.BlockSpec((E, 128), lambda: (0, 0)),
        ],
        out_specs=(
            pl.BlockSpec((8, S), lambda: (0, 0)),
            pl.BlockSpec((8, E), lambda: (0, 0)),
            pl.BlockSpec((S, 256), lambda: (0, 0)),
        ),
        scratch_shapes=[pltpu.VMEM((S, S), jnp.float32)],
    )(x, gate_w, eb)


def _make_dispatch():
    mesh = plsc.VectorSubcoreMesh(core_axis_name="c", subcore_axis_name="s")

    @functools.partial(
        pl.kernel, mesh=mesh,
        out_type=jax.ShapeDtypeStruct((PADTOT, H // 2), jnp.int32),
        scratch_types=[
            pltpu.VMEM((TPW, H // 2), jnp.int32),
            pltpu.VMEM((TPW,), jnp.int32),
            pltpu.VMEM((TPW,), jnp.int32),
            pltpu.SemaphoreType.DMA,
            pltpu.SemaphoreType.DMA,
        ])
    def disp(xb_hbm, dst_hbm, xs_hbm, rows_v, i1_v, i2_v, s0, s1):
        wid = lax.axis_index("s") * NC + lax.axis_index("c")
        base = wid * TPW
        pltpu.sync_copy(xb_hbm.at[pl.ds(base, TPW)], rows_v)
        pltpu.sync_copy(dst_hbm.at[0, pl.ds(base, TPW)], i1_v)
        pltpu.sync_copy(dst_hbm.at[1, pl.ds(base, TPW)], i2_v)
        c0 = pltpu.async_copy(rows_v, xs_hbm.at[i1_v], s0)
        c1 = pltpu.async_copy(rows_v, xs_hbm.at[i2_v], s1)
        c0.wait()
        c1.wait()

    return disp


def _ffn_body(oc_ref, xs_ref, W1a_ref, W1b_ref, W2a_ref, W2b_ref,
              W3a_ref, W3b_ref, ys_ref):
    e = pl.program_id(0)
    off = oc_ref[0, e]
    cnt = oc_ref[1, e]
    trips = (cnt + T - 1) // T
    HI = I // 2

    def tile_body(t, carry):
        base = pl.multiple_of(off + t * T, 8)
        xsf = xs_ref[pl.ds(base, T), :].astype(jnp.float32)
        dn = (((1,), (1,)), ((), ()))
        a = jnp.concatenate([
            jax.lax.dot_general(xsf, W1a_ref[0], dimension_numbers=dn,
                                preferred_element_type=jnp.float32),
            jax.lax.dot_general(xsf, W1b_ref[0], dimension_numbers=dn,
                                preferred_element_type=jnp.float32),
        ], axis=1)
        bb = jnp.concatenate([
            jax.lax.dot_general(xsf, W3a_ref[0], dimension_numbers=dn,
                                preferred_element_type=jnp.float32),
            jax.lax.dot_general(xsf, W3b_ref[0], dimension_numbers=dn,
                                preferred_element_type=jnp.float32),
        ], axis=1)
        hh = a * jax.nn.sigmoid(a) * bb
        y = (jax.lax.dot_general(
                hh[:, :HI], W2a_ref[0], dimension_numbers=dn,
                preferred_element_type=jnp.float32)
             + jax.lax.dot_general(
                hh[:, HI:], W2b_ref[0], dimension_numbers=dn,
                preferred_element_type=jnp.float32))
        ys_ref[pl.ds(base, T), :] = y.astype(jnp.bfloat16)
        return carry

    jax.lax.fori_loop(0, trips, tile_body, 0)


def _ffn(oc, xs, W1, W2, W3):
    HI = I // 2
    return pl.pallas_call(
        _ffn_body,
        grid=(E,),
        out_shape=jax.ShapeDtypeStruct((PADTOT, H), jnp.bfloat16),
        in_specs=[
            pl.BlockSpec(memory_space=pltpu.SMEM),
            pl.BlockSpec((PADTOT, H), lambda e: (0, 0)),
            pl.BlockSpec((1, HI, H), lambda e: (e, 0, 0)),
            pl.BlockSpec((1, HI, H), lambda e: (e, 1, 0)),
            pl.BlockSpec((1, H, HI), lambda e: (e, 0, 0)),
            pl.BlockSpec((1, H, HI), lambda e: (e, 0, 1)),
            pl.BlockSpec((1, HI, H), lambda e: (e, 0, 0)),
            pl.BlockSpec((1, HI, H), lambda e: (e, 1, 0)),
        ],
        out_specs=pl.BlockSpec((PADTOT, H), lambda e: (0, 0)),
        compiler_params=pltpu.CompilerParams(
            dimension_semantics=("arbitrary",),
            vmem_limit_bytes=66060288),
    )(oc, xs, W1, W1, W2, W2, W3, W3)


def _make_gather():
    mesh = plsc.VectorSubcoreMesh(core_axis_name="c", subcore_axis_name="s")

    @functools.partial(
        pl.kernel, mesh=mesh,
        out_type=[
            jax.ShapeDtypeStruct((S, H // 2), jnp.int32),
            jax.ShapeDtypeStruct((S, H // 2), jnp.int32),
        ],
        scratch_types=[
            pltpu.VMEM((TPW,), jnp.int32),
            pltpu.VMEM((TPW,), jnp.int32),
            pltpu.VMEM((TPW, H // 2), jnp.int32),
            pltpu.SemaphoreType.DMA,
        ])
    def gath(ys_hbm, dst_hbm, g1_hbm, g2_hbm, i1_v, i2_v, buf_v, s0):
        wid = lax.axis_index("s") * NC + lax.axis_index("c")
        base = wid * TPW
        pltpu.sync_copy(dst_hbm.at[0, pl.ds(base, TPW)], i1_v)
        pltpu.sync_copy(dst_hbm.at[1, pl.ds(base, TPW)], i2_v)
        pltpu.async_copy(ys_hbm.at[i1_v], buf_v, s0).wait()
        pltpu.sync_copy(buf_v, g1_hbm.at[pl.ds(base, TPW)])
        pltpu.async_copy(ys_hbm.at[i2_v], buf_v, s0).wait()
        pltpu.sync_copy(buf_v, g2_hbm.at[pl.ds(base, TPW)])

    return gath


def _combine_body(g1_ref, g2_ref, wc_ref, out_ref):
    w1 = wc_ref[:, 0:1]
    w2 = wc_ref[:, 128:129]
    g1 = g1_ref[...].astype(jnp.float32)
    g2 = g2_ref[...].astype(jnp.float32)
    out_ref[...] = g1 * w1 + g2 * w2


def _combine(g1, g2, wc):
    return pl.pallas_call(
        _combine_body,
        out_shape=jax.ShapeDtypeStruct((S, H), jnp.float32),
        in_specs=[
            pl.BlockSpec((S, H), lambda: (0, 0)),
            pl.BlockSpec((S, H), lambda: (0, 0)),
            pl.BlockSpec((S, 256), lambda: (0, 0)),
        ],
        out_specs=pl.BlockSpec((S, H), lambda: (0, 0)),
    )(g1, g2, wc)


def kernel(hidden_states, gate_w, e_bias, W1, W2, W3):
    b, s, h = hidden_states.shape
    x = hidden_states.reshape(s, h)
    eb = jnp.broadcast_to(e_bias[:, None], (E, 128))

    dests, oc, wc = _router(x, gate_w, eb)

    xb = x.astype(jnp.bfloat16)
    xbi = jax.lax.bitcast_convert_type(
        xb.reshape(S, H // 2, 2), jnp.int32)
    xsi = _make_dispatch()(xbi, dests)
    xs = jax.lax.bitcast_convert_type(
        xsi, jnp.bfloat16).reshape(PADTOT, H)

    ys = _ffn(oc, xs, W1, W2, W3)
    ysi = jax.lax.bitcast_convert_type(
        ys.reshape(PADTOT, H // 2, 2), jnp.int32)

    g1i, g2i = _make_gather()(ysi, dests)
    g1 = jax.lax.bitcast_convert_type(g1i, jnp.bfloat16).reshape(S, H)
    g2 = jax.lax.bitcast_convert_type(g2i, jnp.bfloat16).reshape(S, H)
    out = _combine(g1, g2, wc)
    return out.reshape(b, s, h)


# speedup vs baseline: 1.2371x; 1.0009x over previous
import functools
import jax
import jax.numpy as jnp
from jax import lax
from jax.experimental import pallas as pl
from jax.experimental.pallas import tpu as pltpu
from jax.experimental.pallas import tpu_sc as plsc

E = 64
TOP_K = 2
H = 1024
I = 1024
S = 2048
T = 128
NC = 2
NS = 16
NW = NC * NS
TPW = S // NW
PADTOT = 4672


def _router_body(x_ref, gw_ref, eb_ref, dst_ref, oc_ref, wc_ref, ut_ref):
    logits = jax.lax.dot_general(
        gw_ref[...], x_ref[...],
        dimension_numbers=(((1,), (1,)), ((), ())),
        preferred_element_type=jnp.float32)
    rw = jax.nn.sigmoid(logits)
    scores = rw + eb_ref[:, 0:1]
    esub = jax.lax.broadcasted_iota(jnp.int32, (E, S), 0)
    m1 = jnp.max(scores, axis=0, keepdims=True)
    a1 = jnp.min(jnp.where(scores == m1, esub, E), axis=0, keepdims=True)
    sel1 = esub == a1
    masked = jnp.where(sel1, -jnp.inf, scores)
    m2 = jnp.max(masked, axis=0, keepdims=True)
    a2 = jnp.min(jnp.where(masked == m2, esub, E), axis=0, keepdims=True)
    sel2 = esub == a2

    C = 512
    for j0 in range(0, S, C):
        ii = jax.lax.broadcasted_iota(jnp.int32, (S, C), 0)
        jj = jax.lax.broadcasted_iota(jnp.int32, (S, C), 1) + j0
        ut_ref[:, j0:j0 + C] = (ii <= jj).astype(jnp.float32)

    self_f = (sel1 | sel2).astype(jnp.float32)
    rank = jax.lax.dot_general(
        self_f, ut_ref[...], dimension_numbers=(((1,), (0,)), ((), ())),
        preferred_element_type=jnp.float32) - 1.0

    ones_row = jnp.ones((1, S), jnp.float32)
    cnt_row = jax.lax.dot_general(
        ones_row, self_f, dimension_numbers=(((1,), (1,)), ((), ())),
        preferred_element_type=jnp.float32)
    pcnt = (((cnt_row.astype(jnp.int32) + 7) // 8) * 8).astype(jnp.float32)
    ei = jax.lax.broadcasted_iota(jnp.int32, (E, E), 0)
    ej = jax.lax.broadcasted_iota(jnp.int32, (E, E), 1)
    slt = (ei < ej).astype(jnp.float32)
    offs_row = jax.lax.dot_general(
        pcnt, slt, dimension_numbers=(((1,), (0,)), ((), ())),
        preferred_element_type=jnp.float32)

    sel1f = sel1.astype(jnp.float32)
    sel2f = sel2.astype(jnp.float32)
    og1 = jax.lax.dot_general(
        offs_row, sel1f, dimension_numbers=(((1,), (0,)), ((), ())),
        preferred_element_type=jnp.float32,
        precision=jax.lax.Precision.HIGHEST)
    og2 = jax.lax.dot_general(
        offs_row, sel2f, dimension_numbers=(((1,), (0,)), ((), ())),
        preferred_element_type=jnp.float32,
        precision=jax.lax.Precision.HIGHEST)
    r1 = jnp.sum(jnp.where(sel1, rank, 0.0), axis=0, keepdims=True)
    r2 = jnp.sum(jnp.where(sel2, rank, 0.0), axis=0, keepdims=True)
    dest1 = og1 + r1
    dest2 = og2 + r2

    di8 = jax.lax.broadcasted_iota(jnp.int32, (8, S), 0)
    dst_ref[...] = jnp.where(
        di8 == 0, dest1, jnp.where(di8 == 1, dest2, 0.0)).astype(jnp.int32)
    de8 = jax.lax.broadcasted_iota(jnp.int32, (8, E), 0)
    oc_ref[...] = jnp.where(
        de8 == 0, offs_row, jnp.where(de8 == 1, cnt_row, 0.0)).astype(jnp.int32)

    ones_e = jnp.ones((E, 128), jnp.float32)
    w1col = jax.lax.dot_general(
        jnp.where(sel1, rw, 0.0), ones_e,
        dimension_numbers=(((0,), (0,)), ((), ())),
        preferred_element_type=jnp.float32)
    w2col = jax.lax.dot_general(
        jnp.where(sel2, rw, 0.0), ones_e,
        dimension_numbers=(((0,), (0,)), ((), ())),
        preferred_element_type=jnp.float32)
    ssum = w1col + w2col
    wc_ref[:, 0:128] = w1col / ssum
    wc_ref[:, 128:256] = w2col / ssum


def _router(x, gate_w, eb):
    return pl.pallas_call(
        _router_body,
        out_shape=(
            jax.ShapeDtypeStruct((8, S), jnp.int32),
            jax.ShapeDtypeStruct((8, E), jnp.int32),
            jax.ShapeDtypeStruct((S, 256), jnp.float32),
        ),
        in_specs=[
            pl.BlockSpec((S, H), lambda: (0, 0)),
            pl.BlockSpec((E, H), lambda: (0, 0)),
            pl.BlockSpec((E, 128), lambda: (0, 0)),
        ],
        out_specs=(
            pl.BlockSpec((8, S), lambda: (0, 0)),
            pl.BlockSpec((8, E), lambda: (0, 0)),
            pl.BlockSpec((S, 256), lambda: (0, 0)),
        ),
        scratch_shapes=[pltpu.VMEM((S, S), jnp.float32)],
    )(x, gate_w, eb)


def _make_dispatch():
    mesh = plsc.VectorSubcoreMesh(core_axis_name="c", subcore_axis_name="s")

    @functools.partial(
        pl.kernel, mesh=mesh,
        compiler_params=pltpu.CompilerParams(use_tc_tiling_on_sc=True),
        out_type=jax.ShapeDtypeStruct((PADTOT, H // 2), jnp.int32),
        scratch_types=[
            pltpu.VMEM((TPW, H // 2), jnp.int32),
            pltpu.VMEM((TPW,), jnp.int32),
            pltpu.VMEM((TPW,), jnp.int32),
            pltpu.SemaphoreType.DMA,
            pltpu.SemaphoreType.DMA,
        ])
    def disp(xb_hbm, dst_hbm, xs_hbm, rows_v, i1_v, i2_v, s0, s1):
        wid = lax.axis_index("s") * NC + lax.axis_index("c")
        base = wid * TPW
        pltpu.sync_copy(xb_hbm.at[pl.ds(base, TPW)], rows_v)
        pltpu.sync_copy(dst_hbm.at[0, pl.ds(base, TPW)], i1_v)
        pltpu.sync_copy(dst_hbm.at[1, pl.ds(base, TPW)], i2_v)
        c0 = pltpu.async_copy(rows_v, xs_hbm.at[i1_v], s0)
        c1 = pltpu.async_copy(rows_v, xs_hbm.at[i2_v], s1)
        c0.wait()
        c1.wait()

    return disp


def _ffn_body(oc_ref, xs_ref, W1a_ref, W1b_ref, W2a_ref, W2b_ref,
              W3a_ref, W3b_ref, ys_ref):
    e = pl.program_id(0)
    off = oc_ref[0, e]
    cnt = oc_ref[1, e]
    trips = (cnt + T - 1) // T
    HI = I // 2

    def tile_body(t, carry):
        base = pl.multiple_of(off + t * T, 8)
        xsf = xs_ref[pl.ds(base, T), :].astype(jnp.float32)
        dn = (((1,), (1,)), ((), ()))
        a = jnp.concatenate([
            jax.lax.dot_general(xsf, W1a_ref[0], dimension_numbers=dn,
                                preferred_element_type=jnp.float32),
            jax.lax.dot_general(xsf, W1b_ref[0], dimension_numbers=dn,
                                preferred_element_type=jnp.float32),
        ], axis=1)
        bb = jnp.concatenate([
            jax.lax.dot_general(xsf, W3a_ref[0], dimension_numbers=dn,
                                preferred_element_type=jnp.float32),
            jax.lax.dot_general(xsf, W3b_ref[0], dimension_numbers=dn,
                                preferred_element_type=jnp.float32),
        ], axis=1)
        hh = a * jax.nn.sigmoid(a) * bb
        y = (jax.lax.dot_general(
                hh[:, :HI], W2a_ref[0], dimension_numbers=dn,
                preferred_element_type=jnp.float32)
             + jax.lax.dot_general(
                hh[:, HI:], W2b_ref[0], dimension_numbers=dn,
                preferred_element_type=jnp.float32))
        ys_ref[pl.ds(base, T), :] = y.astype(jnp.bfloat16)
        return carry

    jax.lax.fori_loop(0, trips, tile_body, 0)


def _ffn(oc, xs, W1, W2, W3):
    HI = I // 2
    return pl.pallas_call(
        _ffn_body,
        grid=(E,),
        out_shape=jax.ShapeDtypeStruct((PADTOT, H), jnp.bfloat16),
        in_specs=[
            pl.BlockSpec(memory_space=pltpu.SMEM),
            pl.BlockSpec((PADTOT, H), lambda e: (0, 0)),
            pl.BlockSpec((1, HI, H), lambda e: (e, 0, 0)),
            pl.BlockSpec((1, HI, H), lambda e: (e, 1, 0)),
            pl.BlockSpec((1, H, HI), lambda e: (e, 0, 0)),
            pl.BlockSpec((1, H, HI), lambda e: (e, 0, 1)),
            pl.BlockSpec((1, HI, H), lambda e: (e, 0, 0)),
            pl.BlockSpec((1, HI, H), lambda e: (e, 1, 0)),
        ],
        out_specs=pl.BlockSpec((PADTOT, H), lambda e: (0, 0)),
        compiler_params=pltpu.CompilerParams(
            dimension_semantics=("arbitrary",),
            vmem_limit_bytes=66060288),
    )(oc, xs, W1, W1, W2, W2, W3, W3)


def _make_gather():
    mesh = plsc.VectorSubcoreMesh(core_axis_name="c", subcore_axis_name="s")

    @functools.partial(
        pl.kernel, mesh=mesh,
        compiler_params=pltpu.CompilerParams(use_tc_tiling_on_sc=True),
        out_type=[
            jax.ShapeDtypeStruct((S, H // 2), jnp.int32),
            jax.ShapeDtypeStruct((S, H // 2), jnp.int32),
        ],
        scratch_types=[
            pltpu.VMEM((TPW,), jnp.int32),
            pltpu.VMEM((TPW,), jnp.int32),
            pltpu.VMEM((TPW, H // 2), jnp.int32),
            pltpu.SemaphoreType.DMA,
        ])
    def gath(ys_hbm, dst_hbm, g1_hbm, g2_hbm, i1_v, i2_v, buf_v, s0):
        wid = lax.axis_index("s") * NC + lax.axis_index("c")
        base = wid * TPW
        pltpu.sync_copy(dst_hbm.at[0, pl.ds(base, TPW)], i1_v)
        pltpu.sync_copy(dst_hbm.at[1, pl.ds(base, TPW)], i2_v)
        pltpu.async_copy(ys_hbm.at[i1_v], buf_v, s0).wait()
        pltpu.sync_copy(buf_v, g1_hbm.at[pl.ds(base, TPW)])
        pltpu.async_copy(ys_hbm.at[i2_v], buf_v, s0).wait()
        pltpu.sync_copy(buf_v, g2_hbm.at[pl.ds(base, TPW)])

    return gath


def _combine_body(g1_ref, g2_ref, wc_ref, out_ref):
    w1 = wc_ref[:, 0:1]
    w2 = wc_ref[:, 128:129]
    g1 = g1_ref[...].astype(jnp.float32)
    g2 = g2_ref[...].astype(jnp.float32)
    out_ref[...] = g1 * w1 + g2 * w2


def _combine(g1, g2, wc):
    return pl.pallas_call(
        _combine_body,
        out_shape=jax.ShapeDtypeStruct((S, H), jnp.float32),
        in_specs=[
            pl.BlockSpec((S, H), lambda: (0, 0)),
            pl.BlockSpec((S, H), lambda: (0, 0)),
            pl.BlockSpec((S, 256), lambda: (0, 0)),
        ],
        out_specs=pl.BlockSpec((S, H), lambda: (0, 0)),
    )(g1, g2, wc)


def kernel(hidden_states, gate_w, e_bias, W1, W2, W3):
    b, s, h = hidden_states.shape
    x = hidden_states.reshape(s, h)
    eb = jnp.broadcast_to(e_bias[:, None], (E, 128))

    dests, oc, wc = _router(x, gate_w, eb)

    xb = x.astype(jnp.bfloat16)
    xbi = jax.lax.bitcast_convert_type(
        xb.reshape(S, H // 2, 2), jnp.int32)
    xsi = _make_dispatch()(xbi, dests)
    xs = jax.lax.bitcast_convert_type(
        xsi, jnp.bfloat16).reshape(PADTOT, H)

    ys = _ffn(oc, xs, W1, W2, W3)
    ysi = jax.lax.bitcast_convert_type(
        ys.reshape(PADTOT, H // 2, 2), jnp.int32)

    g1i, g2i = _make_gather()(ysi, dests)
    g1 = jax.lax.bitcast_convert_type(g1i, jnp.bfloat16).reshape(S, H)
    g2 = jax.lax.bitcast_convert_type(g2i, jnp.bfloat16).reshape(S, H)
    out = _combine(g1, g2, wc)
    return out.reshape(b, s, h)
